# Initial kernel scaffold; baseline (speedup 1.0000x reference)
import jax
import jax.numpy as jnp
from jax.experimental import pallas as pl


def kernel(loc_data, conf_data, prior_data):
    raise NotImplementedError("write your pallas kernel here")


# speedup vs baseline: 6.2804x; 6.2804x over previous
import functools

import jax
import jax.numpy as jnp
from jax import lax
from jax.experimental import pallas as pl
from jax.experimental.pallas import tpu as pltpu
from jax.experimental.pallas import tpu_sc as plsc

NUM_CLASSES = 81
TOP_K = 200
CONF_THRESH = 0.01
VAR0 = 0.1
VAR1 = 0.2
BATCH = 8
P = 20000

L = 16
NW = 32
TASKS = BATCH * NUM_CLASSES
ROW_ITERS = P // L
NBUCKET = 256
HI_CAP = 224
FIN_CAP = 224
GHALF = 112
OUT_ROW = TOP_K * 5
SHIFTS = (18, 10, 2, 0)


def _scan_buckets(hist_ref, acc0):
    def body(k, carry):
        acc, beta, above, cntb = carry
        b = 255 - k
        h = hist_ref[pl.ds(b * L, L)]
        cnt = jnp.sum(h)
        crossed = (acc < TOP_K) & (acc + cnt >= TOP_K)
        beta = jnp.where(crossed, b, beta)
        above = jnp.where(crossed, acc, above)
        cntb = jnp.where(crossed, cnt, cntb)
        return acc + cnt, beta, above, cntb

    _, beta, above, cntb = lax.fori_loop(
        0, NBUCKET, body, (acc0, jnp.int32(0), acc0, jnp.int32(0)))
    return beta, above, cntb


def _clear(ref, n_words):
    z = jnp.zeros((L,), jnp.int32)
    def body(i, _):
        ref[pl.ds(i * L, L)] = z
        return 0
    lax.fori_loop(0, n_words // L, body, 0)


def _clear_f32(ref, n_words):
    z = jnp.zeros((L,), jnp.float32)
    def body(i, _):
        ref[pl.ds(i * L, L)] = z
        return 0
    lax.fori_loop(0, n_words // L, body, 0)


def _body(conf_hbm, comb_hbm, out_hbm,
          scores_v, keys_a, idx_a, keys_b, idx_b, hist_v,
          keys_h, idx_h, keys_f, idx_f, lidx_v,
          rows_v, stage_v, sem):
    lanes = lax.iota(jnp.int32, L)
    ones = jnp.ones((L,), jnp.int32)
    wid = lax.axis_index("s") * 2 + lax.axis_index("c")
    t_start = jnp.minimum(wid, 8) * 21 + jnp.maximum(wid - 8, 0) * 20
    t_end = t_start + jnp.where(wid < 8, 21, 20)

    def run_task(t, _):
        b = t // NUM_CLASSES
        pltpu.sync_copy(conf_hbm.at[t], scores_v)

        _clear(hist_v, NBUCKET * L)
        def hist0(j, _):
            s = scores_v[pl.ds(j * L, L)]
            k = jnp.where(s > CONF_THRESH, lax.bitcast_convert_type(s, jnp.int32), 0)
            bucket = lax.shift_right_logical(k, SHIFTS[0]) & 0xFF
            plsc.addupdate_scatter(hist_v, [bucket * L + lanes], ones)
            return 0
        lax.fori_loop(0, ROW_ITERS, hist0, 0)

        beta, above, cntb = _scan_buckets(hist_v, jnp.int32(0))

        def part0(j, carry):
            hi_off, mid_off = carry
            s = scores_v[pl.ds(j * L, L)]
            k = jnp.where(s > CONF_THRESH, lax.bitcast_convert_type(s, jnp.int32), 0)
            idx = j * L + lanes
            bucket = lax.shift_right_logical(k, SHIFTS[0]) & 0xFF
            m_hi = bucket > beta
            m_mid = bucket == beta
            plsc.store_compressed(keys_h.at[pl.ds(hi_off, L)], k, mask=m_hi)
            plsc.store_compressed(idx_h.at[pl.ds(hi_off, L)], idx, mask=m_hi)
            plsc.store_compressed(keys_a.at[pl.ds(mid_off, L)], k, mask=m_mid)
            plsc.store_compressed(idx_a.at[pl.ds(mid_off, L)], idx, mask=m_mid)
            return (hi_off + jnp.sum(jnp.where(m_hi, 1, 0)),
                    mid_off + jnp.sum(jnp.where(m_mid, 1, 0)))
        hi_n, mid_n = lax.fori_loop(0, ROW_ITERS, part0, (jnp.int32(0), jnp.int32(0)))

        def refine(src_k, src_i, dst_k, dst_i, shift, a_n, m_n):
            n_it = (m_n + (L - 1)) // L
            need = a_n + m_n > TOP_K

            def level(_):
                _clear(hist_v, NBUCKET * L)
                def histj(j, _):
                    k = src_k[pl.ds(j * L, L)]
                    msk = j * L + lanes < m_n
                    bucket = lax.shift_right_logical(k, shift) & 0xFF
                    plsc.addupdate_scatter(hist_v, [bucket * L + lanes], ones,
                                           mask=msk)
                    return 0
                lax.fori_loop(0, n_it, histj, 0)
                beta_l, above_l, _ = _scan_buckets(hist_v, a_n)

                def partj(j, carry):
                    hi_off, mid_off = carry
                    k = src_k[pl.ds(j * L, L)]
                    idx = src_i[pl.ds(j * L, L)]
                    msk = j * L + lanes < m_n
                    bucket = lax.shift_right_logical(k, shift) & 0xFF
                    m_hi = (bucket > beta_l) & msk
                    m_mid = (bucket == beta_l) & msk
                    plsc.store_compressed(keys_h.at[pl.ds(hi_off, L)], k, mask=m_hi)
                    plsc.store_compressed(idx_h.at[pl.ds(hi_off, L)], idx, mask=m_hi)
                    plsc.store_compressed(dst_k.at[pl.ds(mid_off, L)], k, mask=m_mid)
                    plsc.store_compressed(dst_i.at[pl.ds(mid_off, L)], idx, mask=m_mid)
                    return (hi_off + jnp.sum(jnp.where(m_hi, 1, 0)),
                            mid_off + jnp.sum(jnp.where(m_mid, 1, 0)))
                _, new_m = lax.fori_loop(0, n_it, partj, (a_n, jnp.int32(0)))
                return above_l, new_m

            def passthrough(_):
                def cpj(j, _):
                    msk = j * L + lanes < m_n
                    k = src_k[pl.ds(j * L, L)]
                    idx = src_i[pl.ds(j * L, L)]
                    dst_k[pl.ds(j * L, L)] = k
                    dst_i[pl.ds(j * L, L)] = idx
                    return 0
                lax.fori_loop(0, n_it, cpj, 0)
                return a_n, m_n

            return lax.cond(need, level, passthrough, 0)

        a1, m1 = refine(keys_a, idx_a, keys_b, idx_b, SHIFTS[1], above, mid_n)
        a2, m2 = refine(keys_b, idx_b, keys_a, idx_a, SHIFTS[2], a1, m1)
        a3, m3 = refine(keys_a, idx_a, keys_b, idx_b, SHIFTS[3], a2, m2)

        def fin_hi(j, _):
            msk = j * L + lanes < a3
            keys_f[pl.ds(j * L, L)] = jnp.where(msk, keys_h[pl.ds(j * L, L)], 0)
            idx_f[pl.ds(j * L, L)] = jnp.where(msk, idx_h[pl.ds(j * L, L)], 0)
            return 0
        lax.fori_loop(0, FIN_CAP // L, fin_hi, 0)
        n_mid_take = TOP_K - a3
        def fin_mid(j, _):
            pos = j * L + lanes
            msk = pos < n_mid_take
            k = keys_b[pl.ds(j * L, L)]
            idx = idx_b[pl.ds(j * L, L)]
            plsc.store_scatter(keys_f, [a3 + pos], k, mask=msk)
            plsc.store_scatter(idx_f, [a3 + pos], idx, mask=msk)
            return 0
        lax.fori_loop(0, (n_mid_take + (L - 1)) // L, fin_mid, 0)

        for j in range(FIN_CAP // L):
            idx = idx_f[pl.ds(j * L, L)]
            r, col = j // (GHALF // L), (j % (GHALF // L)) * L
            lidx_v[r, pl.ds(col, L)] = idx + b * P
        for r in range(2):
            pltpu.async_copy(comb_hbm.at[lidx_v.at[r]],
                             rows_v.at[pl.ds(r * GHALF, GHALF)], sem).wait()

        _clear_f32(stage_v, OUT_ROW + 8)
        def rank_tile(i, _):
            ki = keys_f[pl.ds(i * L, L)]
            pos_i = i * L + lanes
            def rj(jt, acc):
                kt = keys_f[pl.ds(jt * L, L)]
                base = jt * L
                for l in range(L):
                    kj = kt[l]
                    gt = jnp.where(kj > ki, 1, 0)
                    eqb = jnp.where((kj == ki) & (base + l < pos_i), 1, 0)
                    acc = acc + gt + eqb
                return acc
            rank = lax.fori_loop(0, FIN_CAP // L, rj, jnp.zeros((L,), jnp.int32))

            c0 = jnp.zeros((L,), jnp.int32)
            lx = plsc.load_gather(rows_v, [pos_i, c0])
            ly = plsc.load_gather(rows_v, [pos_i, c0 + 1])
            lw = plsc.load_gather(rows_v, [pos_i, c0 + 2])
            lh = plsc.load_gather(rows_v, [pos_i, c0 + 3])
            px = plsc.load_gather(rows_v, [pos_i, c0 + 4])
            py = plsc.load_gather(rows_v, [pos_i, c0 + 5])
            pw = plsc.load_gather(rows_v, [pos_i, c0 + 6])
            ph = plsc.load_gather(rows_v, [pos_i, c0 + 7])
            cx = px + lx * VAR0 * pw
            cy = py + ly * VAR0 * ph
            w = pw * jnp.exp(lw * VAR1)
            h = ph * jnp.exp(lh * VAR1)
            val = lax.bitcast_convert_type(ki, jnp.float32)
            valid = (val > CONF_THRESH) & (pos_i < TOP_K)
            r5 = rank * 5
            plsc.store_scatter(stage_v, [r5], val, mask=valid)
            plsc.store_scatter(stage_v, [r5 + 1], cx - w * 0.5, mask=valid)
            plsc.store_scatter(stage_v, [r5 + 2], cy - h * 0.5, mask=valid)
            plsc.store_scatter(stage_v, [r5 + 3], cx + w * 0.5, mask=valid)
            plsc.store_scatter(stage_v, [r5 + 4], cy + h * 0.5, mask=valid)
            return 0
        lax.fori_loop(0, FIN_CAP // L, rank_tile, 0)

        pltpu.sync_copy(stage_v.at[pl.ds(0, OUT_ROW)], out_hbm.at[t])
        return 0

    lax.fori_loop(t_start, t_end, run_task, 0)


@jax.jit
def _detect_sc(conf_t, comb):
    mesh = plsc.VectorSubcoreMesh(core_axis_name="c", subcore_axis_name="s",
                                  num_cores=2, num_subcores=16)
    return pl.kernel(
        _body,
        out_type=jax.ShapeDtypeStruct((TASKS, OUT_ROW), jnp.float32),
        mesh=mesh,
        compiler_params=pltpu.CompilerParams(
            needs_layout_passes=False, use_tc_tiling_on_sc=False),
        scratch_types=[
            pltpu.VMEM((P,), jnp.float32),
            pltpu.VMEM((P + L,), jnp.int32),
            pltpu.VMEM((P + L,), jnp.int32),
            pltpu.VMEM((P + L,), jnp.int32),
            pltpu.VMEM((P + L,), jnp.int32),
            pltpu.VMEM((NBUCKET * L,), jnp.int32),
            pltpu.VMEM((HI_CAP,), jnp.int32),
            pltpu.VMEM((HI_CAP,), jnp.int32),
            pltpu.VMEM((FIN_CAP,), jnp.int32),
            pltpu.VMEM((FIN_CAP,), jnp.int32),
            pltpu.VMEM((2, GHALF), jnp.int32),
            pltpu.VMEM((FIN_CAP, 8), jnp.float32),
            pltpu.VMEM((OUT_ROW + 8,), jnp.float32),
            pltpu.SemaphoreType.DMA,
        ],
    )(conf_t, comb)


def kernel(loc_data, conf_data, prior_data):
    conf_t = jnp.transpose(conf_data, (0, 2, 1)).reshape(TASKS, P)
    pb = jnp.broadcast_to(prior_data, (BATCH, P, 4))
    comb = jnp.concatenate([loc_data, pb], axis=-1).reshape(BATCH * P, 8)
    out = _detect_sc(conf_t, comb)
    return out.reshape(BATCH, NUM_CLASSES, TOP_K, 5)
